# baseline (speedup 1.0000x reference)
import jax
import jax.numpy as jnp
from jax import lax
from jax.experimental import pallas as pl
from jax.experimental.pallas import tpu as pltpu

NEG_SLOPE = 0.01
NB = 512


def _lrelu(v):
    return jnp.where(v > 0, v, NEG_SLOPE * v)


def _dot(a, b):
    return lax.dot_general(a, b, (((1,), (0,)), ((), ())),
                           preferred_element_type=jnp.float32)


def _dot_bt(a, b):
    return lax.dot_general(a, b, (((1,), (1,)), ((), ())),
                           preferred_element_type=jnp.float32)


def _fused_kernel(x_ref, w1_ref, w2_ref, a1_ref, b1r_ref, b2r_ref,
                  bf1_ref, w2p_ref, bf2_ref, o_ref, p1_ref):
    b1r = b1r_ref[...]
    for r in range(14):
        p1_ref[88 * r + 84:88 * r + 88] = jnp.zeros((4, NB), jnp.float32)
    for q in range(7):
        y = _dot_bt(w1_ref[0], x_ref[:, q * 128:q * 128 + 256])
        for ci in range(1, 3):
            y = y + _dot_bt(w1_ref[ci],
                            x_ref[:, ci * 1024 + q * 128:
                                     ci * 1024 + q * 128 + 256])
        for e in range(2):
            base = 352 * e
            pw = jnp.maximum(
                jnp.maximum(y[base:base + 84], y[base + 88:base + 172]),
                jnp.maximum(y[base + 176:base + 260],
                            y[base + 264:base + 348]))
            r = 2 * q + e
            p1_ref[88 * r:88 * r + 84] = _lrelu(pw + b1r)

    b2r = b2r_ref[...]
    h = jnp.zeros((8, NB), jnp.float32)
    for r2 in range(5):
        y = _dot(w2_ref[...], p1_ref[176 * r2:176 * r2 + 528])
        pw = jnp.maximum(jnp.maximum(y[0:80], y[80:160]),
                         jnp.maximum(y[160:240], y[240:320]))
        p2 = _lrelu(pw + b2r)
        h = h + _dot(a1_ref[r2], p2)

    h = _lrelu(h + bf1_ref[...])

    z = _dot(w2p_ref[...], h) + bf2_ref[...]
    o_ref[...] = (1.0 / (1.0 + jnp.exp(-z))).astype(o_ref.dtype)


def _forward_impl(packed, x2, out_pad):
    n_pad = x2.shape[0]
    grid_spec = pltpu.PrefetchScalarGridSpec(
        num_scalar_prefetch=0,
        grid=(n_pad // NB,),
        in_specs=[
            pl.BlockSpec((NB, 3072), lambda b: (b, 0)),
            pl.BlockSpec((3, 704, 256), lambda b: (0, 0, 0)),
            pl.BlockSpec((320, 528), lambda b: (0, 0)),
            pl.BlockSpec((5, 8, 80), lambda b: (0, 0, 0)),
            pl.BlockSpec((84, NB), lambda b: (0, 0)),
            pl.BlockSpec((80, NB), lambda b: (0, 0)),
            pl.BlockSpec((8, NB), lambda b: (0, 0)),
            pl.BlockSpec((out_pad, 8), lambda b: (0, 0)),
            pl.BlockSpec((out_pad, NB), lambda b: (0, 0)),
        ],
        out_specs=pl.BlockSpec((out_pad, NB), lambda b: (0, b)),
        scratch_shapes=[
            pltpu.VMEM((1232, NB), jnp.float32),
        ],
    )
    return pl.pallas_call(
        _fused_kernel,
        out_shape=jax.ShapeDtypeStruct((out_pad, n_pad), jnp.float32),
        grid_spec=grid_spec,
        compiler_params=pltpu.CompilerParams(
            dimension_semantics=("parallel",),
            vmem_limit_bytes=64 * 1024 * 1024,
        ),
    )(x2, packed["w1"], packed["w2"], packed["a1"], packed["b1r"],
      packed["b2r"], packed["bf1"], packed["w2p"], packed["bf2"])


_forward = jax.jit(_forward_impl, static_argnames=("out_pad",))


def _band1(w):
    d = jnp.arange(4)[:, None, None, None, None]
    p = jnp.arange(2)[None, :, None, None, None]
    s = jnp.arange(14)[None, None, :, None, None]
    hh = jnp.arange(8)[None, None, None, :, None]
    win = jnp.arange(32)[None, None, None, None, :]
    im = hh - d
    jm = win - (2 * s + p)
    mask = (im >= 0) & (im < 5) & (jm >= 0) & (jm < 5)
    ic = jnp.clip(im, 0, 4)
    jc = jnp.clip(jm, 0, 4)
    g = w[:, :, ic, jc]
    g = jnp.where(mask[None, None], g, 0.0)
    g = g.transpose(1, 2, 3, 0, 4, 5, 6)
    g = g.reshape(3, 4, 2, 84, 256)
    g = jnp.pad(g, ((0, 0), (0, 0), (0, 0), (0, 4), (0, 0)))
    return g.reshape(3, 704, 256)


def _band2(w):
    d = jnp.arange(2)[:, None, None, None, None]
    po = jnp.arange(2)[None, :, None, None, None]
    t = jnp.arange(5)[None, None, :, None, None]
    rb = jnp.arange(6)[None, None, None, :, None]
    s = jnp.arange(14)[None, None, None, None, :]
    im = rb - d
    jm = s - (2 * t + po)
    mask = (im >= 0) & (im < 5) & (jm >= 0) & (jm < 5)
    ic = jnp.clip(im, 0, 4)
    jc = jnp.clip(jm, 0, 4)
    g = w[:, :, ic, jc]
    g = jnp.where(mask[None, None], g, 0.0)
    g = g.transpose(2, 3, 0, 4, 5, 1, 6)
    g = g.reshape(2, 2, 16, 5, 6, 84)
    g = jnp.pad(g, ((0, 0), (0, 0), (0, 0), (0, 0), (0, 0), (0, 4)))
    return g.reshape(320, 528)


def _pack(w_conv1, b_conv1, w_conv2, b_conv2, w_fc1, b_fc1, w_fc2, b_fc2,
          out_pad):
    f32 = jnp.float32
    out_size = w_fc2.shape[0]
    w1b = _band1(jnp.asarray(w_conv1, f32))
    w2b = _band2(jnp.asarray(w_conv2, f32))
    wf1 = jnp.asarray(w_fc1, f32).reshape(5, 16, 5, 5)
    a1 = wf1.transpose(2, 0, 1, 3).reshape(5, 5, 80)
    a1 = jnp.pad(a1, ((0, 0), (0, 3), (0, 0)))
    b1r = jnp.broadcast_to(
        jnp.repeat(jnp.asarray(b_conv1, f32), 14)[:, None], (84, NB))
    b2r = jnp.broadcast_to(
        jnp.repeat(jnp.asarray(b_conv2, f32), 5)[:, None], (80, NB))
    bf1 = jnp.zeros((8, NB), f32).at[:5].set(
        jnp.broadcast_to(jnp.asarray(b_fc1, f32)[:, None], (5, NB)))
    w2p = jnp.zeros((out_pad, 8), f32).at[:out_size, :5].set(
        jnp.asarray(w_fc2, f32))
    bf2 = jnp.zeros((out_pad, NB), f32).at[:out_size].set(
        jnp.broadcast_to(jnp.asarray(b_fc2, f32)[:, None], (out_size, NB)))
    return {"w1": w1b, "w2": w2b, "a1": a1, "b1r": b1r, "b2r": b2r,
            "bf1": bf1, "w2p": w2p, "bf2": bf2}


def kernel(w_conv1, b_conv1, w_conv2, b_conv2, w_fc1, b_fc1, w_fc2, b_fc2, x):
    n = x.shape[0]
    out_size = w_fc2.shape[0]
    out_pad = max(8, ((out_size + 7) // 8) * 8)
    n_pad = ((n + NB - 1) // NB) * NB

    x2 = jnp.asarray(x, jnp.float32)
    if n_pad != n:
        x2 = jnp.pad(x2, ((0, n_pad - n), (0, 0)))

    packed = _pack(w_conv1, b_conv1, w_conv2, b_conv2,
                   w_fc1, b_fc1, w_fc2, b_fc2, out_pad)
    out = _forward(packed, x2, out_pad)
    return out.T[:n, :out_size]


# speedup vs baseline: 5.0078x; 5.0078x over previous
import jax
import jax.numpy as jnp
from jax import lax
from jax.experimental import pallas as pl
from jax.experimental.pallas import tpu as pltpu

NEG_SLOPE = 0.01
NB = 512


def _lrelu(v):
    return jnp.where(v > 0, v, NEG_SLOPE * v)


def _dot(a, b):
    return lax.dot_general(a, b, (((1,), (0,)), ((), ())),
                           preferred_element_type=jnp.float32)


def _dot_bt(a, b):
    return lax.dot_general(a, b, (((1,), (1,)), ((), ())),
                           preferred_element_type=jnp.float32)


def _fused_kernel(x_ref, w1_ref, w2_ref, a1_ref, b1r_ref, b2r_ref,
                  bf1_ref, w2p_ref, bf2_ref, o_ref, p1_ref):
    b1r = b1r_ref[...]
    for r in range(14):
        p1_ref[88 * r + 84:88 * r + 88] = jnp.zeros((4, NB), jnp.float32)
    for q in range(7):
        y = _dot_bt(w1_ref[0], x_ref[:, q * 128:q * 128 + 256])
        for ci in range(1, 3):
            y = y + _dot_bt(w1_ref[ci],
                            x_ref[:, ci * 1024 + q * 128:
                                     ci * 1024 + q * 128 + 256])
        for e in range(2):
            base = 352 * e
            pw = jnp.maximum(
                jnp.maximum(y[base:base + 84], y[base + 88:base + 172]),
                jnp.maximum(y[base + 176:base + 260],
                            y[base + 264:base + 348]))
            r = 2 * q + e
            p1_ref[88 * r:88 * r + 84] = _lrelu(pw + b1r)

    b2r = b2r_ref[...]
    h = jnp.zeros((8, NB), jnp.float32)
    for r2 in range(5):
        y = _dot(w2_ref[...], p1_ref[176 * r2:176 * r2 + 528])
        pw = jnp.maximum(jnp.maximum(y[0:80], y[80:160]),
                         jnp.maximum(y[160:240], y[240:320]))
        p2 = _lrelu(pw + b2r)
        h = h + _dot(a1_ref[r2], p2)

    h = _lrelu(h + bf1_ref[...])

    z = _dot(w2p_ref[...], h) + bf2_ref[...]
    o_ref[...] = (1.0 / (1.0 + jnp.exp(-z))).astype(o_ref.dtype)


def _forward_impl(packed, x2, out_pad):
    n_pad = x2.shape[0]
    grid_spec = pltpu.PrefetchScalarGridSpec(
        num_scalar_prefetch=0,
        grid=(n_pad // NB,),
        in_specs=[
            pl.BlockSpec((NB, 3072), lambda b: (b, 0)),
            pl.BlockSpec((3, 704, 256), lambda b: (0, 0, 0)),
            pl.BlockSpec((320, 528), lambda b: (0, 0)),
            pl.BlockSpec((5, 8, 80), lambda b: (0, 0, 0)),
            pl.BlockSpec((84, NB), lambda b: (0, 0)),
            pl.BlockSpec((80, NB), lambda b: (0, 0)),
            pl.BlockSpec((8, NB), lambda b: (0, 0)),
            pl.BlockSpec((out_pad, 8), lambda b: (0, 0)),
            pl.BlockSpec((out_pad, NB), lambda b: (0, 0)),
        ],
        out_specs=pl.BlockSpec((out_pad, NB), lambda b: (0, b)),
        scratch_shapes=[
            pltpu.VMEM((1232, NB), jnp.float32),
        ],
    )
    return pl.pallas_call(
        _fused_kernel,
        out_shape=jax.ShapeDtypeStruct((out_pad, n_pad), jnp.float32),
        grid_spec=grid_spec,
        compiler_params=pltpu.CompilerParams(
            dimension_semantics=("parallel",),
            vmem_limit_bytes=64 * 1024 * 1024,
        ),
    )(x2, packed["w1"], packed["w2"], packed["a1"], packed["b1r"],
      packed["b2r"], packed["bf1"], packed["w2p"], packed["bf2"])


_forward = jax.jit(_forward_impl, static_argnames=("out_pad",))


def _band1(w):
    f32 = jnp.float32
    d = jnp.arange(4)
    hh = jnp.arange(8)
    i = jnp.arange(5)
    a1h = (hh[None, :, None] == d[:, None, None] + i[None, None, :])
    p = jnp.arange(2)
    s = jnp.arange(14)
    j = jnp.arange(5)
    win = jnp.arange(32)
    b1w = (win[None, None, None, :] ==
           2 * s[None, :, None, None] + p[:, None, None, None]
           + j[None, None, :, None])
    g = jnp.einsum('kcij,dhi,psjw->cdpkshw',
                   w, a1h.astype(f32), b1w.astype(f32))
    g = g.reshape(3, 4, 2, 84, 256)
    g = jnp.pad(g, ((0, 0), (0, 0), (0, 0), (0, 4), (0, 0)))
    return g.reshape(3, 704, 256)


def _band2(w):
    f32 = jnp.float32
    d = jnp.arange(2)
    rb = jnp.arange(6)
    i = jnp.arange(5)
    a2h = (rb[None, :, None] == d[:, None, None] + i[None, None, :])
    po = jnp.arange(2)
    t = jnp.arange(5)
    j = jnp.arange(5)
    s = jnp.arange(14)
    b2w = (s[None, None, None, :] ==
           2 * t[None, :, None, None] + po[:, None, None, None]
           + j[None, None, :, None])
    g = jnp.einsum('kcij,dri,ptjs->dpktrcs',
                   w, a2h.astype(f32), b2w.astype(f32))
    g = g.reshape(2, 2, 16, 5, 6, 84)
    g = jnp.pad(g, ((0, 0), (0, 0), (0, 0), (0, 0), (0, 0), (0, 4)))
    return g.reshape(320, 528)


def _pack(w_conv1, b_conv1, w_conv2, b_conv2, w_fc1, b_fc1, w_fc2, b_fc2,
          out_pad):
    f32 = jnp.float32
    out_size = w_fc2.shape[0]
    w1b = _band1(jnp.asarray(w_conv1, f32))
    w2b = _band2(jnp.asarray(w_conv2, f32))
    wf1 = jnp.asarray(w_fc1, f32).reshape(5, 16, 5, 5)
    a1 = wf1.transpose(2, 0, 1, 3).reshape(5, 5, 80)
    a1 = jnp.pad(a1, ((0, 0), (0, 3), (0, 0)))
    b1r = jnp.broadcast_to(
        jnp.repeat(jnp.asarray(b_conv1, f32), 14)[:, None], (84, NB))
    b2r = jnp.broadcast_to(
        jnp.repeat(jnp.asarray(b_conv2, f32), 5)[:, None], (80, NB))
    bf1 = jnp.zeros((8, NB), f32).at[:5].set(
        jnp.broadcast_to(jnp.asarray(b_fc1, f32)[:, None], (5, NB)))
    w2p = jnp.zeros((out_pad, 8), f32).at[:out_size, :5].set(
        jnp.asarray(w_fc2, f32))
    bf2 = jnp.zeros((out_pad, NB), f32).at[:out_size].set(
        jnp.broadcast_to(jnp.asarray(b_fc2, f32)[:, None], (out_size, NB)))
    return {"w1": w1b, "w2": w2b, "a1": a1, "b1r": b1r, "b2r": b2r,
            "bf1": bf1, "w2p": w2p, "bf2": bf2}


def kernel(w_conv1, b_conv1, w_conv2, b_conv2, w_fc1, b_fc1, w_fc2, b_fc2, x):
    n = x.shape[0]
    out_size = w_fc2.shape[0]
    out_pad = max(8, ((out_size + 7) // 8) * 8)
    n_pad = ((n + NB - 1) // NB) * NB

    x2 = jnp.asarray(x, jnp.float32)
    if n_pad != n:
        x2 = jnp.pad(x2, ((0, n_pad - n), (0, 0)))

    packed = _pack(w_conv1, b_conv1, w_conv2, b_conv2,
                   w_fc1, b_fc1, w_fc2, b_fc2, out_pad)
    out = _forward(packed, x2, out_pad)
    return out.T[:n, :out_size]


# speedup vs baseline: 5.0406x; 1.0065x over previous
import jax
import jax.numpy as jnp
from jax import lax
from jax.experimental import pallas as pl
from jax.experimental.pallas import tpu as pltpu

NEG_SLOPE = 0.01
NB = 256


def _lrelu(v):
    return jnp.where(v > 0, v, NEG_SLOPE * v)


def _dot(a, b):
    return lax.dot_general(a, b, (((1,), (0,)), ((), ())),
                           preferred_element_type=jnp.float32)


def _dot_bt(a, b):
    return lax.dot_general(a, b, (((1,), (1,)), ((), ())),
                           preferred_element_type=jnp.float32)


def _fused_kernel(x_ref, w1_ref, w2_ref, a1_ref, b1r_ref, b2r_ref,
                  bf1_ref, w2p_ref, bf2_ref, o_ref, p1_ref):
    b1r = b1r_ref[...]
    for r in range(14):
        p1_ref[88 * r + 84:88 * r + 88] = jnp.zeros((4, NB), jnp.float32)
    for q in range(7):
        y = _dot_bt(w1_ref[0], x_ref[:, q * 128:q * 128 + 256])
        for ci in range(1, 3):
            y = y + _dot_bt(w1_ref[ci],
                            x_ref[:, ci * 1024 + q * 128:
                                     ci * 1024 + q * 128 + 256])
        for e in range(2):
            base = 352 * e
            pw = jnp.maximum(
                jnp.maximum(y[base:base + 84], y[base + 88:base + 172]),
                jnp.maximum(y[base + 176:base + 260],
                            y[base + 264:base + 348]))
            r = 2 * q + e
            p1_ref[88 * r:88 * r + 84] = _lrelu(pw + b1r)

    b2r = b2r_ref[...]
    h = jnp.zeros((8, NB), jnp.float32)
    for r2 in range(5):
        y = _dot(w2_ref[...], p1_ref[176 * r2:176 * r2 + 528])
        pw = jnp.maximum(jnp.maximum(y[0:80], y[80:160]),
                         jnp.maximum(y[160:240], y[240:320]))
        p2 = _lrelu(pw + b2r)
        h = h + _dot(a1_ref[r2], p2)

    h = _lrelu(h + bf1_ref[...])

    z = _dot(w2p_ref[...], h) + bf2_ref[...]
    o_ref[...] = (1.0 / (1.0 + jnp.exp(-z))).astype(o_ref.dtype)


def _forward_impl(packed, x2, out_pad):
    n_pad = x2.shape[0]
    grid_spec = pltpu.PrefetchScalarGridSpec(
        num_scalar_prefetch=0,
        grid=(n_pad // NB,),
        in_specs=[
            pl.BlockSpec((NB, 3072), lambda b: (b, 0)),
            pl.BlockSpec((3, 704, 256), lambda b: (0, 0, 0)),
            pl.BlockSpec((320, 528), lambda b: (0, 0)),
            pl.BlockSpec((5, 8, 80), lambda b: (0, 0, 0)),
            pl.BlockSpec((84, NB), lambda b: (0, 0)),
            pl.BlockSpec((80, NB), lambda b: (0, 0)),
            pl.BlockSpec((8, NB), lambda b: (0, 0)),
            pl.BlockSpec((out_pad, 8), lambda b: (0, 0)),
            pl.BlockSpec((out_pad, NB), lambda b: (0, 0)),
        ],
        out_specs=pl.BlockSpec((out_pad, NB), lambda b: (0, b)),
        scratch_shapes=[
            pltpu.VMEM((1232, NB), jnp.float32),
        ],
    )
    return pl.pallas_call(
        _fused_kernel,
        out_shape=jax.ShapeDtypeStruct((out_pad, n_pad), jnp.float32),
        grid_spec=grid_spec,
        compiler_params=pltpu.CompilerParams(
            dimension_semantics=("parallel",),
            vmem_limit_bytes=64 * 1024 * 1024,
        ),
    )(x2, packed["w1"], packed["w2"], packed["a1"], packed["b1r"],
      packed["b2r"], packed["bf1"], packed["w2p"], packed["bf2"])


_forward = jax.jit(_forward_impl, static_argnames=("out_pad",))


def _band1(w):
    f32 = jnp.float32
    d = jnp.arange(4)
    hh = jnp.arange(8)
    i = jnp.arange(5)
    a1h = (hh[None, :, None] == d[:, None, None] + i[None, None, :])
    p = jnp.arange(2)
    s = jnp.arange(14)
    j = jnp.arange(5)
    win = jnp.arange(32)
    b1w = (win[None, None, None, :] ==
           2 * s[None, :, None, None] + p[:, None, None, None]
           + j[None, None, :, None])
    g = jnp.einsum('kcij,dhi,psjw->cdpkshw',
                   w, a1h.astype(f32), b1w.astype(f32))
    g = g.reshape(3, 4, 2, 84, 256)
    g = jnp.pad(g, ((0, 0), (0, 0), (0, 0), (0, 4), (0, 0)))
    return g.reshape(3, 704, 256)


def _band2(w):
    f32 = jnp.float32
    d = jnp.arange(2)
    rb = jnp.arange(6)
    i = jnp.arange(5)
    a2h = (rb[None, :, None] == d[:, None, None] + i[None, None, :])
    po = jnp.arange(2)
    t = jnp.arange(5)
    j = jnp.arange(5)
    s = jnp.arange(14)
    b2w = (s[None, None, None, :] ==
           2 * t[None, :, None, None] + po[:, None, None, None]
           + j[None, None, :, None])
    g = jnp.einsum('kcij,dri,ptjs->dpktrcs',
                   w, a2h.astype(f32), b2w.astype(f32))
    g = g.reshape(2, 2, 16, 5, 6, 84)
    g = jnp.pad(g, ((0, 0), (0, 0), (0, 0), (0, 0), (0, 0), (0, 4)))
    return g.reshape(320, 528)


def _pack(w_conv1, b_conv1, w_conv2, b_conv2, w_fc1, b_fc1, w_fc2, b_fc2,
          out_pad):
    f32 = jnp.float32
    out_size = w_fc2.shape[0]
    w1b = _band1(jnp.asarray(w_conv1, f32))
    w2b = _band2(jnp.asarray(w_conv2, f32))
    wf1 = jnp.asarray(w_fc1, f32).reshape(5, 16, 5, 5)
    a1 = wf1.transpose(2, 0, 1, 3).reshape(5, 5, 80)
    a1 = jnp.pad(a1, ((0, 0), (0, 3), (0, 0)))
    b1r = jnp.broadcast_to(
        jnp.repeat(jnp.asarray(b_conv1, f32), 14)[:, None], (84, NB))
    b2r = jnp.broadcast_to(
        jnp.repeat(jnp.asarray(b_conv2, f32), 5)[:, None], (80, NB))
    bf1 = jnp.zeros((8, NB), f32).at[:5].set(
        jnp.broadcast_to(jnp.asarray(b_fc1, f32)[:, None], (5, NB)))
    w2p = jnp.zeros((out_pad, 8), f32).at[:out_size, :5].set(
        jnp.asarray(w_fc2, f32))
    bf2 = jnp.zeros((out_pad, NB), f32).at[:out_size].set(
        jnp.broadcast_to(jnp.asarray(b_fc2, f32)[:, None], (out_size, NB)))
    return {"w1": w1b, "w2": w2b, "a1": a1, "b1r": b1r, "b2r": b2r,
            "bf1": bf1, "w2p": w2p, "bf2": bf2}


def kernel(w_conv1, b_conv1, w_conv2, b_conv2, w_fc1, b_fc1, w_fc2, b_fc2, x):
    n = x.shape[0]
    out_size = w_fc2.shape[0]
    out_pad = max(8, ((out_size + 7) // 8) * 8)
    n_pad = ((n + NB - 1) // NB) * NB

    x2 = jnp.asarray(x, jnp.float32)
    if n_pad != n:
        x2 = jnp.pad(x2, ((0, n_pad - n), (0, 0)))

    packed = _pack(w_conv1, b_conv1, w_conv2, b_conv2,
                   w_fc1, b_fc1, w_fc2, b_fc2, out_pad)
    out = _forward(packed, x2, out_pad)
    return out.T[:n, :out_size]


# speedup vs baseline: 5.1253x; 1.0168x over previous
import jax
import jax.numpy as jnp
from jax import lax
from jax.experimental import pallas as pl
from jax.experimental.pallas import tpu as pltpu

NEG_SLOPE = 0.01
NB = 256


def _lrelu(v):
    return jnp.where(v > 0, v, NEG_SLOPE * v)


def _dot(a, b):
    return lax.dot_general(a, b, (((1,), (0,)), ((), ())),
                           preferred_element_type=jnp.float32)


def _dot_bt(a, b):
    return lax.dot_general(a, b, (((1,), (1,)), ((), ())),
                           preferred_element_type=jnp.float32)


def _fused_kernel(x_ref, w1_ref, w2_ref, a1_ref, b1r_ref, b2r_ref,
                  bf1_ref, w2p_ref, bf2_ref, o_ref, p1_ref, p2_ref):
    b1r = b1r_ref[...]
    for r in range(14):
        p1_ref[88 * r + 84:88 * r + 88] = jnp.zeros((4, NB), jnp.float32)
    for q in range(7):
        y = _dot_bt(w1_ref[0], x_ref[:, q * 128:q * 128 + 256])
        for ci in range(1, 3):
            y = y + _dot_bt(w1_ref[ci],
                            x_ref[:, ci * 1024 + q * 128:
                                     ci * 1024 + q * 128 + 256])
        for e in range(2):
            base = 352 * e
            pw = jnp.maximum(
                jnp.maximum(y[base:base + 84], y[base + 88:base + 172]),
                jnp.maximum(y[base + 176:base + 260],
                            y[base + 264:base + 348]))
            r = 2 * q + e
            p1_ref[88 * r:88 * r + 84] = _lrelu(pw + b1r)

    b2r = b2r_ref[...]
    for r2 in range(5):
        y = _dot(w2_ref[...], p1_ref[176 * r2:176 * r2 + 528])
        pw = jnp.maximum(jnp.maximum(y[0:80], y[80:160]),
                         jnp.maximum(y[160:240], y[240:320]))
        p2_ref[80 * r2:80 * r2 + 80] = _lrelu(pw + b2r)

    h = _dot(a1_ref[...], p2_ref[...])
    h = _lrelu(h + bf1_ref[...])

    z = _dot(w2p_ref[...], h) + bf2_ref[...]
    o_ref[...] = (1.0 / (1.0 + jnp.exp(-z))).astype(o_ref.dtype)


def _forward_impl(packed, x2, out_pad):
    n_pad = x2.shape[0]
    grid_spec = pltpu.PrefetchScalarGridSpec(
        num_scalar_prefetch=0,
        grid=(n_pad // NB,),
        in_specs=[
            pl.BlockSpec((NB, 3072), lambda b: (b, 0)),
            pl.BlockSpec((3, 704, 256), lambda b: (0, 0, 0)),
            pl.BlockSpec((320, 528), lambda b: (0, 0)),
            pl.BlockSpec((8, 400), lambda b: (0, 0)),
            pl.BlockSpec((84, NB), lambda b: (0, 0)),
            pl.BlockSpec((80, NB), lambda b: (0, 0)),
            pl.BlockSpec((8, NB), lambda b: (0, 0)),
            pl.BlockSpec((out_pad, 8), lambda b: (0, 0)),
            pl.BlockSpec((out_pad, NB), lambda b: (0, 0)),
        ],
        out_specs=pl.BlockSpec((out_pad, NB), lambda b: (0, b)),
        scratch_shapes=[
            pltpu.VMEM((1232, NB), jnp.float32),
            pltpu.VMEM((400, NB), jnp.float32),
        ],
    )
    return pl.pallas_call(
        _fused_kernel,
        out_shape=jax.ShapeDtypeStruct((out_pad, n_pad), jnp.float32),
        grid_spec=grid_spec,
        compiler_params=pltpu.CompilerParams(
            dimension_semantics=("parallel",),
            vmem_limit_bytes=64 * 1024 * 1024,
        ),
    )(x2, packed["w1"], packed["w2"], packed["a1"], packed["b1r"],
      packed["b2r"], packed["bf1"], packed["w2p"], packed["bf2"])


_forward = jax.jit(_forward_impl, static_argnames=("out_pad",))


def _band1(w):
    f32 = jnp.float32
    d = jnp.arange(4)
    hh = jnp.arange(8)
    i = jnp.arange(5)
    a1h = (hh[None, :, None] == d[:, None, None] + i[None, None, :])
    p = jnp.arange(2)
    s = jnp.arange(14)
    j = jnp.arange(5)
    win = jnp.arange(32)
    b1w = (win[None, None, None, :] ==
           2 * s[None, :, None, None] + p[:, None, None, None]
           + j[None, None, :, None])
    g = jnp.einsum('kcij,dhi,psjw->cdpkshw',
                   w, a1h.astype(f32), b1w.astype(f32))
    g = g.reshape(3, 4, 2, 84, 256)
    g = jnp.pad(g, ((0, 0), (0, 0), (0, 0), (0, 4), (0, 0)))
    return g.reshape(3, 704, 256)


def _band2(w):
    f32 = jnp.float32
    d = jnp.arange(2)
    rb = jnp.arange(6)
    i = jnp.arange(5)
    a2h = (rb[None, :, None] == d[:, None, None] + i[None, None, :])
    po = jnp.arange(2)
    t = jnp.arange(5)
    j = jnp.arange(5)
    s = jnp.arange(14)
    b2w = (s[None, None, None, :] ==
           2 * t[None, :, None, None] + po[:, None, None, None]
           + j[None, None, :, None])
    g = jnp.einsum('kcij,dri,ptjs->dpktrcs',
                   w, a2h.astype(f32), b2w.astype(f32))
    g = g.reshape(2, 2, 16, 5, 6, 84)
    g = jnp.pad(g, ((0, 0), (0, 0), (0, 0), (0, 0), (0, 0), (0, 4)))
    return g.reshape(320, 528)


def _pack(w_conv1, b_conv1, w_conv2, b_conv2, w_fc1, b_fc1, w_fc2, b_fc2,
          out_pad):
    f32 = jnp.float32
    out_size = w_fc2.shape[0]
    w1b = _band1(jnp.asarray(w_conv1, f32))
    w2b = _band2(jnp.asarray(w_conv2, f32))
    wf1 = jnp.asarray(w_fc1, f32).reshape(5, 16, 5, 5)
    a1 = wf1.transpose(0, 2, 1, 3).reshape(5, 400)
    a1 = jnp.pad(a1, ((0, 3), (0, 0)))
    b1r = jnp.broadcast_to(
        jnp.repeat(jnp.asarray(b_conv1, f32), 14)[:, None], (84, NB))
    b2r = jnp.broadcast_to(
        jnp.repeat(jnp.asarray(b_conv2, f32), 5)[:, None], (80, NB))
    bf1 = jnp.zeros((8, NB), f32).at[:5].set(
        jnp.broadcast_to(jnp.asarray(b_fc1, f32)[:, None], (5, NB)))
    w2p = jnp.zeros((out_pad, 8), f32).at[:out_size, :5].set(
        jnp.asarray(w_fc2, f32))
    bf2 = jnp.zeros((out_pad, NB), f32).at[:out_size].set(
        jnp.broadcast_to(jnp.asarray(b_fc2, f32)[:, None], (out_size, NB)))
    return {"w1": w1b, "w2": w2b, "a1": a1, "b1r": b1r, "b2r": b2r,
            "bf1": bf1, "w2p": w2p, "bf2": bf2}


def kernel(w_conv1, b_conv1, w_conv2, b_conv2, w_fc1, b_fc1, w_fc2, b_fc2, x):
    n = x.shape[0]
    out_size = w_fc2.shape[0]
    out_pad = max(8, ((out_size + 7) // 8) * 8)
    n_pad = ((n + NB - 1) // NB) * NB

    x2 = jnp.asarray(x, jnp.float32)
    if n_pad != n:
        x2 = jnp.pad(x2, ((0, n_pad - n), (0, 0)))

    packed = _pack(w_conv1, b_conv1, w_conv2, b_conv2,
                   w_fc1, b_fc1, w_fc2, b_fc2, out_pad)
    out = _forward(packed, x2, out_pad)
    return out.T[:n, :out_size]


# speedup vs baseline: 6.4949x; 1.2672x over previous
import jax
import jax.numpy as jnp
from jax import lax
from jax.experimental import pallas as pl
from jax.experimental.pallas import tpu as pltpu

NEG_SLOPE = 0.01
NB = 256


def _lrelu(v):
    return jnp.where(v > 0, v, NEG_SLOPE * v)


def _dot(a, b):
    return lax.dot_general(a, b, (((1,), (0,)), ((), ())),
                           preferred_element_type=jnp.float32)


def _dot_bt(a, b):
    return lax.dot_general(a, b, (((1,), (1,)), ((), ())),
                           preferred_element_type=jnp.float32)


def _fused_kernel(x_ref, w1_ref, w2_ref, a1_ref, b1r_ref, b2r_ref,
                  bf1_ref, w2p_ref, bf2_ref, o_ref, p1_ref, p2_ref):
    b1r = b1r_ref[...]
    for r in range(14):
        p1_ref[88 * r + 84:88 * r + 88] = jnp.zeros((4, NB), jnp.float32)
    for q in range(7):
        y = _dot_bt(w1_ref[0], x_ref[:, q * 128:q * 128 + 256])
        for ci in range(1, 3):
            y = y + _dot_bt(w1_ref[ci],
                            x_ref[:, ci * 1024 + q * 128:
                                     ci * 1024 + q * 128 + 256])
        for e in range(2):
            base = 352 * e
            pw = jnp.maximum(
                jnp.maximum(y[base:base + 84], y[base + 88:base + 172]),
                jnp.maximum(y[base + 176:base + 260],
                            y[base + 264:base + 348]))
            r = 2 * q + e
            p1_ref[88 * r:88 * r + 84] = _lrelu(pw + b1r)

    b2r = b2r_ref[...]
    for r2 in range(5):
        y = _dot(w2_ref[...], p1_ref[176 * r2:176 * r2 + 528])
        pw = jnp.maximum(jnp.maximum(y[0:80], y[80:160]),
                         jnp.maximum(y[160:240], y[240:320]))
        p2_ref[80 * r2:80 * r2 + 80] = _lrelu(pw + b2r)

    h = _dot(a1_ref[...], p2_ref[...])
    h = _lrelu(h + bf1_ref[...])

    z = _dot(w2p_ref[...], h) + bf2_ref[...]
    o_ref[...] = (1.0 / (1.0 + jnp.exp(-z))).astype(o_ref.dtype)


def _forward_impl(packed, x2, out_pad):
    n_pad = x2.shape[0]
    grid_spec = pltpu.PrefetchScalarGridSpec(
        num_scalar_prefetch=0,
        grid=(n_pad // NB,),
        in_specs=[
            pl.BlockSpec((NB, 3072), lambda b: (b, 0)),
            pl.BlockSpec((3, 704, 256), lambda b: (0, 0, 0)),
            pl.BlockSpec((320, 528), lambda b: (0, 0)),
            pl.BlockSpec((8, 400), lambda b: (0, 0)),
            pl.BlockSpec((84, NB), lambda b: (0, 0)),
            pl.BlockSpec((80, NB), lambda b: (0, 0)),
            pl.BlockSpec((8, NB), lambda b: (0, 0)),
            pl.BlockSpec((out_pad, 8), lambda b: (0, 0)),
            pl.BlockSpec((out_pad, NB), lambda b: (0, 0)),
        ],
        out_specs=pl.BlockSpec((out_pad, NB), lambda b: (0, b)),
        scratch_shapes=[
            pltpu.VMEM((1232, NB), jnp.float32),
            pltpu.VMEM((400, NB), jnp.float32),
        ],
    )
    return pl.pallas_call(
        _fused_kernel,
        out_shape=jax.ShapeDtypeStruct((out_pad, n_pad), jnp.float32),
        grid_spec=grid_spec,
        compiler_params=pltpu.CompilerParams(
            dimension_semantics=("parallel",),
            vmem_limit_bytes=64 * 1024 * 1024,
        ),
    )(x2, packed["w1"], packed["w2"], packed["a1"], packed["b1r"],
      packed["b2r"], packed["bf1"], packed["w2p"], packed["bf2"])


_forward = jax.jit(_forward_impl, static_argnames=("out_pad",))


def _band1(w):
    v = jnp.pad(w, ((0, 0), (0, 0), (0, 0), (0, 27)))
    v = v.reshape(6, 3, 160)[:, :, :133].transpose(1, 0, 2)
    bases = []
    for d in range(4):
        for p in range(2):
            off = d * 32 + p
            bases.append(jnp.pad(v, ((0, 0), (0, 0), (off, 125 - off))))
    b = jnp.stack(bases)
    b = b.reshape(4, 2, 3, 6, 258).transpose(2, 0, 1, 3, 4)
    t = jnp.tile(b, (1, 1, 1, 1, 14))[..., :3584]
    g = t.reshape(3, 4, 2, 84, 256)
    g = jnp.pad(g, ((0, 0), (0, 0), (0, 0), (0, 4), (0, 0)))
    return g.reshape(3, 704, 256)


def _band2(w):
    v = jnp.pad(w, ((0, 0), (0, 0), (0, 0), (0, 83)))
    v = v.reshape(16, 6, 440)[:, :, :357]
    vb = jnp.stack([jnp.pad(v[:, ci], ((0, 0), (ci * 14, 70 - ci * 14)))
                    for ci in range(6)], axis=1)
    vs = vb.sum(axis=1)
    bases = []
    for d in range(2):
        for po in range(2):
            off = d * 88 + po
            bases.append(jnp.pad(vs, ((0, 0), (off, 103 - off))))
    b = jnp.stack(bases).reshape(2, 2, 16, 530)
    t = jnp.tile(b, (1, 1, 1, 5))[..., :2640]
    return t.reshape(2, 2, 16, 5, 528).reshape(320, 528)


def _pack(w_conv1, b_conv1, w_conv2, b_conv2, w_fc1, b_fc1, w_fc2, b_fc2,
          out_pad):
    f32 = jnp.float32
    out_size = w_fc2.shape[0]
    w1b = _band1(jnp.asarray(w_conv1, f32))
    w2b = _band2(jnp.asarray(w_conv2, f32))
    wf1 = jnp.asarray(w_fc1, f32).reshape(5, 16, 5, 5)
    a1 = wf1.transpose(0, 2, 1, 3).reshape(5, 400)
    a1 = jnp.pad(a1, ((0, 3), (0, 0)))
    b1r = jnp.broadcast_to(
        jnp.repeat(jnp.asarray(b_conv1, f32), 14)[:, None], (84, NB))
    b2r = jnp.broadcast_to(
        jnp.repeat(jnp.asarray(b_conv2, f32), 5)[:, None], (80, NB))
    bf1 = jnp.zeros((8, NB), f32).at[:5].set(
        jnp.broadcast_to(jnp.asarray(b_fc1, f32)[:, None], (5, NB)))
    w2p = jnp.zeros((out_pad, 8), f32).at[:out_size, :5].set(
        jnp.asarray(w_fc2, f32))
    bf2 = jnp.zeros((out_pad, NB), f32).at[:out_size].set(
        jnp.broadcast_to(jnp.asarray(b_fc2, f32)[:, None], (out_size, NB)))
    return {"w1": w1b, "w2": w2b, "a1": a1, "b1r": b1r, "b2r": b2r,
            "bf1": bf1, "w2p": w2p, "bf2": bf2}


def kernel(w_conv1, b_conv1, w_conv2, b_conv2, w_fc1, b_fc1, w_fc2, b_fc2, x):
    n = x.shape[0]
    out_size = w_fc2.shape[0]
    out_pad = max(8, ((out_size + 7) // 8) * 8)
    n_pad = ((n + NB - 1) // NB) * NB

    x2 = jnp.asarray(x, jnp.float32)
    if n_pad != n:
        x2 = jnp.pad(x2, ((0, n_pad - n), (0, 0)))

    packed = _pack(w_conv1, b_conv1, w_conv2, b_conv2,
                   w_fc1, b_fc1, w_fc2, b_fc2, out_pad)
    out = _forward(packed, x2, out_pad)
    return out.T[:n, :out_size]
